# baseline (speedup 1.0000x reference)
import functools

import jax
import jax.numpy as jnp
from jax import lax
from jax.experimental import pallas as pl
from jax.experimental.pallas import tpu as pltpu
from jax.experimental.pallas import tpu_sc as plsc

N_OTHER = 2000
N_USER = 20000
N_ITEM = 28000
N = N_OTHER + N_USER + N_ITEM
NPAD = 50048
E = 800000
EPAD = 819200
VOCAB = 100000
D = 64
KD = 128
T = 20
B = 4096
NEG = 5

NC = 2
NS = 16
NWK = NC * NS

f32 = jnp.float32
i32 = jnp.int32

_SC_MESH = dict(core_axis_name="c", subcore_axis_name="s")


EMB_ROWS = NPAD * T
EMB_PW = EMB_ROWS // NWK
EMB_G = 1360
EMB_CH = EMB_PW // EMB_G


@functools.partial(
    pl.kernel,
    mesh=plsc.VectorSubcoreMesh(**_SC_MESH),
    compiler_params=pltpu.CompilerParams(use_tc_tiling_on_sc=False),
    out_type=jax.ShapeDtypeStruct((EMB_ROWS, D), f32),
    scratch_types=[
        pltpu.VMEM((EMB_G,), i32),
        pltpu.VMEM((EMB_G, D), f32),
        pltpu.SemaphoreType.DMA,
    ],
)
def _emb_gather(tab, idx, out, idx_v, rows_v, sem):
    wid = lax.axis_index("s") * NC + lax.axis_index("c")

    def body(j, carry):
        base = wid * EMB_PW + j * EMB_G
        pltpu.sync_copy(idx.at[pl.ds(base, EMB_G)], idx_v)
        pltpu.async_copy(tab.at[idx_v], rows_v, sem).wait()
        pltpu.sync_copy(rows_v, out.at[pl.ds(base, EMB_G)])
        return carry

    lax.fori_loop(0, EMB_CH, body, 0)


NPASS = 3
RNG = 16704
NPO = NPASS * RNG
TRASH = RNG
ACCR = RNG + 8
GE = 320
CB = GE + 64
CBLK = CB // 64
PTE = EPAD // NS
GCH2 = PTE // (2 * GE)
CPT = RNG // NS
ZFULL = CPT // GE
ZREM = CPT - ZFULL * GE


def _gcn_scratch():
    return [
        pltpu.VMEM((GE,), i32),
        pltpu.VMEM((GE,), i32),
        pltpu.VMEM((GE,), f32),
        pltpu.VMEM((CB,), i32),
        pltpu.VMEM((CB,), i32),
        pltpu.VMEM((CB,), f32),
        pltpu.VMEM((CBLK, D), i32),
        pltpu.VMEM((CB, D), f32),
    ]


@functools.partial(
    pl.kernel,
    mesh=plsc.VectorSubcoreMesh(**_SC_MESH),
    compiler_params=pltpu.CompilerParams(
        use_tc_tiling_on_sc=False, needs_layout_passes=False),
    out_type=[
        jax.ShapeDtypeStruct((NPO, D), f32),
        jax.ShapeDtypeStruct((NPO, D), f32),
    ],
    scratch_types=_gcn_scratch() + _gcn_scratch() + [
        pltpu.VMEM_SHARED((ACCR, D), f32),
        pltpu.SemaphoreType.DMA,
        pltpu.SemaphoreType.DMA,
        pltpu.SemaphoreType.DMA,
        pltpu.SemaphoreType.DMA,
    ],
)
def _gcn_edges(sup_lo, sup_hi, src, dst, w, out_lo, out_hi,
               srcA, dstA, wA, csrcA, csidxA, cwA, crowA, rowsA,
               srcB, dstB, wB, csrcB, csidxB, cwB, crowB, rowsB,
               acc, sgA, ssA, sgB, ssB):
    c = lax.axis_index("c")
    s = lax.axis_index("s")

    def half(sup, out):
        def stage_compact(base, lo, src_v, dst_v, w_v, csrc_v, csidx_v, cw_v):
            pltpu.sync_copy(src.at[pl.ds(base, GE)], src_v)
            pltpu.sync_copy(dst.at[pl.ds(base, GE)], dst_v)
            pltpu.sync_copy(w.at[pl.ds(base, GE)], w_v)

            def cg(g, cnt):
                s16 = src_v[pl.ds(g * 16, 16)]
                d16 = dst_v[pl.ds(g * 16, 16)]
                w16 = w_v[pl.ds(g * 16, 16)]
                l16 = d16 - lo
                m = (l16 >= 0) & (l16 < RNG)
                lane = lax.iota(i32, 16)
                key = jnp.where(m, lane, lane + 16)
                _, perm = plsc.sort_key_val(key, lane)
                csrc_v[pl.ds(cnt, 16)] = s16.at[perm].get(
                    mode="promise_in_bounds")
                csidx_v[pl.ds(cnt, 16)] = l16.at[perm].get(
                    mode="promise_in_bounds")
                cw_v[pl.ds(cnt, 16)] = w16.at[perm].get(
                    mode="promise_in_bounds")
                return cnt + jnp.sum(m.astype(i32))

            cnt = lax.fori_loop(0, GE // 16, cg, 0)
            for k in range(4):
                csrc_v[pl.ds(cnt + k * 16, 16)] = jnp.zeros((16,), i32)
                csidx_v[pl.ds(cnt + k * 16, 16)] = jnp.full((16,), TRASH, i32)
                cw_v[pl.ds(cnt + k * 16, 16)] = jnp.zeros((16,), f32)
            return (cnt + 63) // 64

        def gathers(nblk, csrc_v, rows_v, sem, start):
            for b in range(CBLK):
                cp = pltpu.make_async_copy(
                    sup.at[csrc_v.at[pl.ds(b * 64, 64)]],
                    rows_v.at[pl.ds(b * 64, 64)], sem)

                @pl.when(b < nblk)
                def _(cp=cp):
                    if start:
                        cp.start()
                    else:
                        cp.wait()

        def mul(nblk, cw_v, rows_v):
            def mg(g, cr):
                w16 = cw_v[pl.ds(g * 16, 16)]
                for jj in range(16):
                    r = g * 16 + jj
                    wspl = w16.at[jnp.full((16,), jj, i32)].get(
                        mode="promise_in_bounds")
                    for k in range(4):
                        rows_v[r, pl.ds(k * 16, 16)] = (
                            rows_v[r, pl.ds(k * 16, 16)] * wspl)
                return cr

            lax.fori_loop(0, nblk * 4, mg, 0)

        def scatters(nblk, csidx_v, crow_v, rows_v, sem, start):
            for b in range(CBLK):
                if start:
                    for k in range(4):
                        crow_v[b, pl.ds(k * 16, 16)] = (
                            csidx_v[pl.ds(b * 64 + k * 16, 16)])
                cp = pltpu.make_async_copy(
                    rows_v.at[pl.ds(b * 64, 64)],
                    acc.at[crow_v.at[b]], sem)

                @pl.when(b < nblk)
                def _(cp=cp):
                    if start:
                        cp.start(add=True)
                    else:
                        cp.wait()

        def do_pass(p, carry):
            lo = p * RNG

            def zb(r, cr):
                for k in range(4):
                    rowsA[r, pl.ds(k * 16, 16)] = jnp.zeros((16,), f32)
                return cr

            lax.fori_loop(0, GE, zb, 0)
            for q in range(ZFULL):
                pltpu.sync_copy(rowsA.at[pl.ds(0, GE)],
                                acc.at[pl.ds(s * CPT + q * GE, GE)])
            pltpu.sync_copy(rowsA.at[pl.ds(0, ZREM)],
                            acc.at[pl.ds(s * CPT + ZFULL * GE, ZREM)])
            plsc.subcore_barrier()

            def pair(i, cr):
                baseA = s * PTE + (2 * i) * GE
                nA = stage_compact(baseA, lo, srcA, dstA, wA,
                                   csrcA, csidxA, cwA)
                gathers(nA, csrcA, rowsA, sgA, True)
                nB = stage_compact(baseA + GE, lo, srcB, dstB, wB,
                                   csrcB, csidxB, cwB)
                gathers(nA, csrcA, rowsA, sgA, False)
                mul(nA, cwA, rowsA)
                scatters(nA, csidxA, crowA, rowsA, ssA, True)
                gathers(nB, csrcB, rowsB, sgB, True)
                gathers(nB, csrcB, rowsB, sgB, False)
                mul(nB, cwB, rowsB)
                scatters(nB, csidxB, crowB, rowsB, ssB, True)
                scatters(nA, csidxA, crowA, rowsA, ssA, False)
                scatters(nB, csidxB, crowB, rowsB, ssB, False)
                return cr

            lax.fori_loop(0, GCH2, pair, 0)
            plsc.subcore_barrier()
            pltpu.sync_copy(acc.at[pl.ds(s * CPT, CPT)],
                            out.at[pl.ds(lo + s * CPT, CPT)])
            plsc.subcore_barrier()
            return carry

        lax.fori_loop(0, NPASS, do_pass, 0)

    @pl.when(c == 0)
    def _():
        half(sup_lo, out_lo)

    @pl.when(c == 1)
    def _():
        half(sup_hi, out_hi)


GB = (NEG + 2) * B
GPW = GB // NWK


@functools.partial(
    pl.kernel,
    mesh=plsc.VectorSubcoreMesh(**_SC_MESH),
    compiler_params=pltpu.CompilerParams(use_tc_tiling_on_sc=False),
    out_type=[jax.ShapeDtypeStruct((GB, D), f32) for _ in range(4)],
    scratch_types=[
        pltpu.VMEM((GPW,), i32),
        pltpu.VMEM((GPW, D), f32),
        pltpu.SemaphoreType.DMA,
    ],
)
def _final_gather(t1l, t1h, t2l, t2h, gidx, o1l, o1h, o2l, o2h,
                  idx_v, rows_v, sem):
    wid = lax.axis_index("s") * NC + lax.axis_index("c")
    base = wid * GPW
    pltpu.sync_copy(gidx.at[pl.ds(base, GPW)], idx_v)
    for tab, out in ((t1l, o1l), (t1h, o1h), (t2l, o2l), (t2h, o2h)):
        pltpu.async_copy(tab.at[idx_v], rows_v, sem).wait()
        pltpu.sync_copy(rows_v, out.at[pl.ds(base, GPW)])


BN = 1472
NB = NPAD // BN


def _gru_body(x_ref, wi0, wh0, bi0, bh0, wi1, wh1, bi1, bh1, out_ref, y0):
    x = x_ref[...]
    h = jnp.zeros((BN, D), f32)
    for t in range(T):
        gi = jnp.dot(x[:, t, :], wi0[...], preferred_element_type=f32) + bi0[...]
        gh = jnp.dot(h, wh0[...], preferred_element_type=f32) + bh0[...]
        r = jax.nn.sigmoid(gi[:, :D] + gh[:, :D])
        z = jax.nn.sigmoid(gi[:, D:2 * D] + gh[:, D:2 * D])
        n = jnp.tanh(gi[:, 2 * D:] + r * gh[:, 2 * D:])
        h = (1.0 - z) * n + z * h
        y0[t] = h
    h = jnp.zeros((BN, D), f32)
    for t in range(T):
        gi = jnp.dot(y0[t], wi1[...], preferred_element_type=f32) + bi1[...]
        gh = jnp.dot(h, wh1[...], preferred_element_type=f32) + bh1[...]
        r = jax.nn.sigmoid(gi[:, :D] + gh[:, :D])
        z = jax.nn.sigmoid(gi[:, D:2 * D] + gh[:, D:2 * D])
        n = jnp.tanh(gi[:, 2 * D:] + r * gh[:, 2 * D:])
        h = (1.0 - z) * n + z * h
    out_ref[...] = h


def _rep(shape):
    return pl.BlockSpec(shape, lambda i: tuple(0 for _ in shape))


def _gru_call(emb3, wi0, wh0, bi0, bh0, wi1, wh1, bi1, bh1):
    return pl.pallas_call(
        _gru_body,
        grid=(NB,),
        in_specs=[
            pl.BlockSpec((BN, T, D), lambda i: (i, 0, 0)),
            _rep((D, 3 * D)), _rep((D, 3 * D)), _rep((1, 3 * D)), _rep((1, 3 * D)),
            _rep((D, 3 * D)), _rep((D, 3 * D)), _rep((1, 3 * D)), _rep((1, 3 * D)),
        ],
        out_specs=pl.BlockSpec((BN, D), lambda i: (i, 0)),
        out_shape=jax.ShapeDtypeStruct((NPAD, D), f32),
        scratch_shapes=[pltpu.VMEM((T, BN, D), f32)],
    )(emb3, wi0, wh0, bi0, bh0, wi1, wh1, bi1, bh1)


def _sup1_body(pe_ref, h_ref, w_ref, olo_ref, ohi_ref):
    x = pe_ref[...] + h_ref[...]
    sup = jnp.dot(x, w_ref[...], preferred_element_type=f32)
    olo_ref[...] = sup[:, :D]
    ohi_ref[...] = sup[:, D:]


def _sup1_call(pos_pad, h1, w1):
    return pl.pallas_call(
        _sup1_body,
        grid=(NB,),
        in_specs=[
            pl.BlockSpec((BN, D), lambda i: (i, 0)),
            pl.BlockSpec((BN, D), lambda i: (i, 0)),
            _rep((D, KD)),
        ],
        out_specs=[pl.BlockSpec((BN, D), lambda i: (i, 0))] * 2,
        out_shape=[jax.ShapeDtypeStruct((NPAD, D), f32)] * 2,
    )(pos_pad, h1, w1)


def _sup2_body(alo_ref, ahi_ref, w_ref, b1_ref, olo_ref, ohi_ref):
    clo = alo_ref[...] + b1_ref[...][:, :D]
    chi = ahi_ref[...] + b1_ref[...][:, D:]
    sup = (jnp.dot(clo, w_ref[...][:D], preferred_element_type=f32)
           + jnp.dot(chi, w_ref[...][D:], preferred_element_type=f32))
    olo_ref[...] = sup[:, :D]
    ohi_ref[...] = sup[:, D:]


def _sup2_call(a1lo, a1hi, w2, b1r):
    return pl.pallas_call(
        _sup2_body,
        grid=(NB,),
        in_specs=[
            pl.BlockSpec((BN, D), lambda i: (i, 0)),
            pl.BlockSpec((BN, D), lambda i: (i, 0)),
            _rep((KD, KD)),
            _rep((1, KD)),
        ],
        out_specs=[pl.BlockSpec((BN, D), lambda i: (i, 0))] * 2,
        out_shape=[jax.ShapeDtypeStruct((NPAD, D), f32)] * 2,
    )(a1lo, a1hi, w2, b1r)


LB = 512
NLB = B // LB


def _loss_body(u1l, u1h, u2l, u2h, p1l, p1h, p2l, p2h,
               n1l, n1h, n2l, n2h, wl, bl, b1, b2, out_ref):
    i = pl.program_id(0)
    wlv = wl[...]
    b1v = b1[...]
    b2v = b2[...]

    def emb(lo1, hi1, lo2, hi2):
        return (jnp.dot(lo1 + b1v[:, :D], wlv[0:D], preferred_element_type=f32)
                + jnp.dot(hi1 + b1v[:, D:], wlv[D:2 * D], preferred_element_type=f32)
                + jnp.dot(lo2 + b2v[:, :D], wlv[2 * D:3 * D], preferred_element_type=f32)
                + jnp.dot(hi2 + b2v[:, D:], wlv[3 * D:], preferred_element_type=f32)
                + bl[...])

    u = emb(u1l[...], u1h[...], u2l[...], u2h[...])
    pv = emb(p1l[...], p1h[...], p2l[...], p2h[...])
    nv = emb(n1l[...].reshape(NEG * LB, D), n1h[...].reshape(NEG * LB, D),
             n2l[...].reshape(NEG * LB, D), n2h[...].reshape(NEG * LB, D))
    nv = nv.reshape(NEG, LB, 2 * D)
    pos_score = jnp.sum(u * pv, axis=1)
    neg_score = jnp.sum(nv * u[None, :, :], axis=2)
    diff = neg_score - pos_score[None, :] + 1.0
    part = jnp.sum(jnp.clip(jnp.mean(diff, axis=0), 1e-06, 10000.0))
    out_ref[...] = jnp.where(i == 0, 0.0, out_ref[...]) + part


def _loss_call(gs, wl, blr, b1r, b2r):
    urow = pl.BlockSpec((LB, D), lambda i: (i, 0))
    prow = pl.BlockSpec((LB, D), lambda i: (i, 0))
    nrow = pl.BlockSpec((NEG, LB, D), lambda i: (0, i, 0))
    return pl.pallas_call(
        _loss_body,
        grid=(NLB,),
        in_specs=([urow] * 4 + [prow] * 4 + [nrow] * 4
                  + [_rep((2 * KD, 2 * D)), _rep((1, 2 * D)),
                     _rep((1, KD)), _rep((1, KD))]),
        out_specs=pl.BlockSpec((1, 1), lambda i: (0, 0)),
        out_shape=jax.ShapeDtypeStruct((1, 1), f32),
    )(*gs, wl, blr, b1r, b2r)


def kernel(user, pos_item, neg_item, label, lookup_index, edge_index,
           edge_weight, word_emb, other_pos, user_pos, item_pos,
           W_ih0, W_hh0, b_ih0, b_hh0, W_ih1, W_hh1, b_ih1, b_hh1,
           W1, b1, W2, b2, Wl, bl):
    wt = word_emb.at[0].set(0.0)
    idx_pad = jnp.concatenate([
        lookup_index.reshape(-1).astype(i32),
        jnp.zeros((EMB_ROWS - N * T,), i32)])
    emb = _emb_gather(wt, idx_pad)
    emb3 = emb.reshape(NPAD, T, D)

    h1 = _gru_call(emb3, W_ih0.T, W_hh0.T, b_ih0.reshape(1, -1),
                   b_hh0.reshape(1, -1), W_ih1.T, W_hh1.T,
                   b_ih1.reshape(1, -1), b_hh1.reshape(1, -1))

    pos_pad = jnp.concatenate(
        [other_pos, user_pos, item_pos, jnp.zeros((NPAD - N, D), f32)], axis=0)
    sup1_lo, sup1_hi = _sup1_call(pos_pad, h1, W1)

    zpad = jnp.zeros((EPAD - E,), i32)
    srcp = jnp.concatenate([edge_index[0].astype(i32), zpad])
    dstp = jnp.concatenate([edge_index[1].astype(i32), zpad])
    wp = jnp.concatenate([edge_weight, jnp.zeros((EPAD - E,), f32)])

    a1lo, a1hi = _gcn_edges(sup1_lo, sup1_hi, srcp, dstp, wp)
    s2lo, s2hi = _sup2_call(a1lo, a1hi, W2, b1.reshape(1, -1))
    a2lo, a2hi = _gcn_edges(s2lo, s2hi, srcp, dstp, wp)

    gidx = jnp.concatenate([
        user + N_OTHER,
        pos_item + N_OTHER + N_USER,
        neg_item + N_OTHER + N_USER]).astype(i32)
    g1l, g1h, g2l, g2h = _final_gather(a1lo, a1hi, a2lo, a2hi, gidx)

    def parts(g):
        return g[:B], g[B:2 * B], g[2 * B:].reshape(NEG, B, D)

    u1l, p1l, n1l = parts(g1l)
    u1h, p1h, n1h = parts(g1h)
    u2l, p2l, n2l = parts(g2l)
    u2h, p2h, n2h = parts(g2h)
    gs = (u1l, u1h, u2l, u2h, p1l, p1h, p2l, p2h, n1l, n1h, n2l, n2h)
    out = _loss_call(gs, Wl, bl.reshape(1, -1), b1.reshape(1, -1),
                     b2.reshape(1, -1))
    return out[0, 0]


# speedup vs baseline: 1.0380x; 1.0380x over previous
import functools

import jax
import jax.numpy as jnp
from jax import lax
from jax.experimental import pallas as pl
from jax.experimental.pallas import tpu as pltpu
from jax.experimental.pallas import tpu_sc as plsc

N_OTHER = 2000
N_USER = 20000
N_ITEM = 28000
N = N_OTHER + N_USER + N_ITEM
NPAD = 50048
E = 800000
EPAD = 819200
VOCAB = 100000
D = 64
KD = 128
T = 20
B = 4096
NEG = 5

NC = 2
NS = 16
NWK = NC * NS

f32 = jnp.float32
i32 = jnp.int32

_SC_MESH = dict(core_axis_name="c", subcore_axis_name="s")


EMB_ROWS = NPAD * T
EMB_PW = EMB_ROWS // NWK
EMB_G = 1360
EMB_CH = EMB_PW // EMB_G


@functools.partial(
    pl.kernel,
    mesh=plsc.VectorSubcoreMesh(**_SC_MESH),
    compiler_params=pltpu.CompilerParams(use_tc_tiling_on_sc=False),
    out_type=jax.ShapeDtypeStruct((EMB_ROWS, D), f32),
    scratch_types=[
        pltpu.VMEM((EMB_G,), i32),
        pltpu.VMEM((EMB_G, D), f32),
        pltpu.SemaphoreType.DMA,
    ],
)
def _emb_gather(tab, idx, out, idx_v, rows_v, sem):
    wid = lax.axis_index("s") * NC + lax.axis_index("c")

    def body(j, carry):
        base = wid * EMB_PW + j * EMB_G
        pltpu.sync_copy(idx.at[pl.ds(base, EMB_G)], idx_v)
        pltpu.async_copy(tab.at[idx_v], rows_v, sem).wait()
        pltpu.sync_copy(rows_v, out.at[pl.ds(base, EMB_G)])
        return carry

    lax.fori_loop(0, EMB_CH, body, 0)


NPASS = 2
RNG = NPAD // 2
NPO = NPASS * RNG
TRASH = RNG
ACCR = RNG + 8
GE = 320
CB = GE + 64
CBLK = CB // 64
PTE = EPAD // NS
GCH2 = PTE // (2 * GE)
CPT = RNG // NS
ZFULL = CPT // CB
ZREM = CPT - ZFULL * CB


def _gcn_scratch():
    return [
        pltpu.VMEM((GE,), i32),
        pltpu.VMEM((GE,), i32),
        pltpu.VMEM((GE,), f32),
        pltpu.VMEM((CB,), i32),
        pltpu.VMEM((CB,), i32),
        pltpu.VMEM((CB,), f32),
        pltpu.VMEM((CBLK, D), i32),
        pltpu.VMEM((CB, D), f32),
        pltpu.VMEM((CB, D), jnp.bfloat16),
    ]


@functools.partial(
    pl.kernel,
    mesh=plsc.VectorSubcoreMesh(**_SC_MESH),
    compiler_params=pltpu.CompilerParams(
        use_tc_tiling_on_sc=False, needs_layout_passes=False),
    out_type=[
        jax.ShapeDtypeStruct((NPO, D), jnp.bfloat16),
        jax.ShapeDtypeStruct((NPO, D), jnp.bfloat16),
    ],
    scratch_types=_gcn_scratch() + _gcn_scratch() + [
        pltpu.VMEM_SHARED((ACCR, D), jnp.bfloat16),
        pltpu.SemaphoreType.DMA,
        pltpu.SemaphoreType.DMA,
        pltpu.SemaphoreType.DMA,
        pltpu.SemaphoreType.DMA,
    ],
)
def _gcn_edges(sup_lo, sup_hi, src, dst, w, out_lo, out_hi,
               srcA, dstA, wA, csrcA, csidxA, cwA, crowA, rowsA, bfA,
               srcB, dstB, wB, csrcB, csidxB, cwB, crowB, rowsB, bfB,
               acc, sgA, ssA, sgB, ssB):
    c = lax.axis_index("c")
    s = lax.axis_index("s")

    def half(sup, out):
        def stage_compact(base, lo, src_v, dst_v, w_v, csrc_v, csidx_v, cw_v):
            pltpu.sync_copy(src.at[pl.ds(base, GE)], src_v)
            pltpu.sync_copy(dst.at[pl.ds(base, GE)], dst_v)
            pltpu.sync_copy(w.at[pl.ds(base, GE)], w_v)

            def cg(g, cnt):
                s16 = src_v[pl.ds(g * 16, 16)]
                d16 = dst_v[pl.ds(g * 16, 16)]
                w16 = w_v[pl.ds(g * 16, 16)]
                l16 = d16 - lo
                m = (l16 >= 0) & (l16 < RNG)
                lane = lax.iota(i32, 16)
                key = jnp.where(m, lane, lane + 16)
                _, perm = plsc.sort_key_val(key, lane)
                csrc_v[pl.ds(cnt, 16)] = s16.at[perm].get(
                    mode="promise_in_bounds")
                csidx_v[pl.ds(cnt, 16)] = l16.at[perm].get(
                    mode="promise_in_bounds")
                cw_v[pl.ds(cnt, 16)] = w16.at[perm].get(
                    mode="promise_in_bounds")
                return cnt + jnp.sum(m.astype(i32))

            cnt = lax.fori_loop(0, GE // 16, cg, 0)
            for k in range(4):
                csrc_v[pl.ds(cnt + k * 16, 16)] = jnp.zeros((16,), i32)
                csidx_v[pl.ds(cnt + k * 16, 16)] = jnp.full((16,), TRASH, i32)
                cw_v[pl.ds(cnt + k * 16, 16)] = jnp.zeros((16,), f32)
            return (cnt + 63) // 64

        def gathers(nblk, csrc_v, rows_v, sem, start):
            for b in range(CBLK):
                cp = pltpu.make_async_copy(
                    sup.at[csrc_v.at[pl.ds(b * 64, 64)]],
                    rows_v.at[pl.ds(b * 64, 64)], sem)

                @pl.when(b < nblk)
                def _(cp=cp):
                    if start:
                        cp.start()
                    else:
                        cp.wait()

        def mul(nblk, cw_v, rows_v, bf_v):
            def mg(g, cr):
                w16 = cw_v[pl.ds(g * 16, 16)]
                for jj in range(16):
                    r = g * 16 + jj
                    wspl = w16.at[jnp.full((16,), jj, i32)].get(
                        mode="promise_in_bounds")
                    m0 = rows_v[r, pl.ds(0, 16)] * wspl
                    m1 = rows_v[r, pl.ds(16, 16)] * wspl
                    m2 = rows_v[r, pl.ds(32, 16)] * wspl
                    m3 = rows_v[r, pl.ds(48, 16)] * wspl
                    bf_v[r, pl.ds(0, 32)] = plsc.pack(
                        m0, m1, format=plsc.PackFormat.INTERLEAVED)
                    bf_v[r, pl.ds(32, 32)] = plsc.pack(
                        m2, m3, format=plsc.PackFormat.INTERLEAVED)
                return cr

            lax.fori_loop(0, nblk * 4, mg, 0)

        def scatters(nblk, csidx_v, crow_v, bf_v, sem, start):
            for b in range(CBLK):
                if start:
                    for k in range(4):
                        crow_v[b, pl.ds(k * 16, 16)] = (
                            csidx_v[pl.ds(b * 64 + k * 16, 16)])
                cp = pltpu.make_async_copy(
                    bf_v.at[pl.ds(b * 64, 64)],
                    acc.at[crow_v.at[b]], sem)

                @pl.when(b < nblk)
                def _(cp=cp):
                    if start:
                        cp.start(add=True)
                    else:
                        cp.wait()

        def do_pass(p, carry):
            lo = p * RNG

            def zb(r, cr):
                for k in range(2):
                    bfA[r, pl.ds(k * 32, 32)] = jnp.zeros((32,), jnp.bfloat16)
                return cr

            lax.fori_loop(0, CB, zb, 0)
            for q in range(ZFULL):
                pltpu.sync_copy(bfA.at[pl.ds(0, CB)],
                                acc.at[pl.ds(s * CPT + q * CB, CB)])
            pltpu.sync_copy(bfA.at[pl.ds(0, ZREM)],
                            acc.at[pl.ds(s * CPT + ZFULL * CB, ZREM)])
            plsc.subcore_barrier()

            def pair(i, cr):
                baseA = s * PTE + (2 * i) * GE
                nA = stage_compact(baseA, lo, srcA, dstA, wA,
                                   csrcA, csidxA, cwA)
                gathers(nA, csrcA, rowsA, sgA, True)
                nB = stage_compact(baseA + GE, lo, srcB, dstB, wB,
                                   csrcB, csidxB, cwB)
                gathers(nA, csrcA, rowsA, sgA, False)
                mul(nA, cwA, rowsA, bfA)
                scatters(nA, csidxA, crowA, bfA, ssA, True)
                gathers(nB, csrcB, rowsB, sgB, True)
                gathers(nB, csrcB, rowsB, sgB, False)
                mul(nB, cwB, rowsB, bfB)
                scatters(nB, csidxB, crowB, bfB, ssB, True)
                scatters(nA, csidxA, crowA, bfA, ssA, False)
                scatters(nB, csidxB, crowB, bfB, ssB, False)
                return cr

            lax.fori_loop(0, GCH2, pair, 0)
            plsc.subcore_barrier()
            pltpu.sync_copy(acc.at[pl.ds(s * CPT, CPT)],
                            out.at[pl.ds(lo + s * CPT, CPT)])
            plsc.subcore_barrier()
            return carry

        lax.fori_loop(0, NPASS, do_pass, 0)

    @pl.when(c == 0)
    def _():
        half(sup_lo, out_lo)

    @pl.when(c == 1)
    def _():
        half(sup_hi, out_hi)


GB = (NEG + 2) * B
GPW = GB // NWK


@functools.partial(
    pl.kernel,
    mesh=plsc.VectorSubcoreMesh(**_SC_MESH),
    compiler_params=pltpu.CompilerParams(use_tc_tiling_on_sc=False),
    out_type=[jax.ShapeDtypeStruct((GB, D), jnp.bfloat16) for _ in range(4)],
    scratch_types=[
        pltpu.VMEM((GPW,), i32),
        pltpu.VMEM((GPW, D), jnp.bfloat16),
        pltpu.SemaphoreType.DMA,
    ],
)
def _final_gather(t1l, t1h, t2l, t2h, gidx, o1l, o1h, o2l, o2h,
                  idx_v, rows_v, sem):
    wid = lax.axis_index("s") * NC + lax.axis_index("c")
    base = wid * GPW
    pltpu.sync_copy(gidx.at[pl.ds(base, GPW)], idx_v)
    for tab, out in ((t1l, o1l), (t1h, o1h), (t2l, o2l), (t2h, o2h)):
        pltpu.async_copy(tab.at[idx_v], rows_v, sem).wait()
        pltpu.sync_copy(rows_v, out.at[pl.ds(base, GPW)])


BN = 1472
NB = NPAD // BN


def _gru_body(x_ref, wi0, wh0, bi0, bh0, wi1, wh1, bi1, bh1, out_ref, y0):
    x = x_ref[...]
    h = jnp.zeros((BN, D), f32)
    for t in range(T):
        gi = jnp.dot(x[:, t, :], wi0[...], preferred_element_type=f32) + bi0[...]
        gh = jnp.dot(h, wh0[...], preferred_element_type=f32) + bh0[...]
        r = jax.nn.sigmoid(gi[:, :D] + gh[:, :D])
        z = jax.nn.sigmoid(gi[:, D:2 * D] + gh[:, D:2 * D])
        n = jnp.tanh(gi[:, 2 * D:] + r * gh[:, 2 * D:])
        h = (1.0 - z) * n + z * h
        y0[t] = h
    h = jnp.zeros((BN, D), f32)
    for t in range(T):
        gi = jnp.dot(y0[t], wi1[...], preferred_element_type=f32) + bi1[...]
        gh = jnp.dot(h, wh1[...], preferred_element_type=f32) + bh1[...]
        r = jax.nn.sigmoid(gi[:, :D] + gh[:, :D])
        z = jax.nn.sigmoid(gi[:, D:2 * D] + gh[:, D:2 * D])
        n = jnp.tanh(gi[:, 2 * D:] + r * gh[:, 2 * D:])
        h = (1.0 - z) * n + z * h
    out_ref[...] = h


def _rep(shape):
    return pl.BlockSpec(shape, lambda i: tuple(0 for _ in shape))


def _gru_call(emb3, wi0, wh0, bi0, bh0, wi1, wh1, bi1, bh1):
    return pl.pallas_call(
        _gru_body,
        grid=(NB,),
        in_specs=[
            pl.BlockSpec((BN, T, D), lambda i: (i, 0, 0)),
            _rep((D, 3 * D)), _rep((D, 3 * D)), _rep((1, 3 * D)), _rep((1, 3 * D)),
            _rep((D, 3 * D)), _rep((D, 3 * D)), _rep((1, 3 * D)), _rep((1, 3 * D)),
        ],
        out_specs=pl.BlockSpec((BN, D), lambda i: (i, 0)),
        out_shape=jax.ShapeDtypeStruct((NPAD, D), f32),
        scratch_shapes=[pltpu.VMEM((T, BN, D), f32)],
    )(emb3, wi0, wh0, bi0, bh0, wi1, wh1, bi1, bh1)


def _sup1_body(pe_ref, h_ref, w_ref, olo_ref, ohi_ref):
    x = pe_ref[...] + h_ref[...]
    sup = jnp.dot(x, w_ref[...], preferred_element_type=f32)
    olo_ref[...] = sup[:, :D]
    ohi_ref[...] = sup[:, D:]


def _sup1_call(pos_pad, h1, w1):
    return pl.pallas_call(
        _sup1_body,
        grid=(NB,),
        in_specs=[
            pl.BlockSpec((BN, D), lambda i: (i, 0)),
            pl.BlockSpec((BN, D), lambda i: (i, 0)),
            _rep((D, KD)),
        ],
        out_specs=[pl.BlockSpec((BN, D), lambda i: (i, 0))] * 2,
        out_shape=[jax.ShapeDtypeStruct((NPAD, D), f32)] * 2,
    )(pos_pad, h1, w1)


def _sup2_body(alo_ref, ahi_ref, w_ref, b1_ref, olo_ref, ohi_ref):
    clo = alo_ref[...].astype(f32) + b1_ref[...][:, :D]
    chi = ahi_ref[...].astype(f32) + b1_ref[...][:, D:]
    sup = (jnp.dot(clo, w_ref[...][:D], preferred_element_type=f32)
           + jnp.dot(chi, w_ref[...][D:], preferred_element_type=f32))
    olo_ref[...] = sup[:, :D]
    ohi_ref[...] = sup[:, D:]


def _sup2_call(a1lo, a1hi, w2, b1r):
    return pl.pallas_call(
        _sup2_body,
        grid=(NB,),
        in_specs=[
            pl.BlockSpec((BN, D), lambda i: (i, 0)),
            pl.BlockSpec((BN, D), lambda i: (i, 0)),
            _rep((KD, KD)),
            _rep((1, KD)),
        ],
        out_specs=[pl.BlockSpec((BN, D), lambda i: (i, 0))] * 2,
        out_shape=[jax.ShapeDtypeStruct((NPAD, D), f32)] * 2,
    )(a1lo, a1hi, w2, b1r)


LB = 512
NLB = B // LB


def _loss_body(u1l, u1h, u2l, u2h, p1l, p1h, p2l, p2h,
               n1l, n1h, n2l, n2h, wl, bl, b1, b2, out_ref):
    i = pl.program_id(0)
    wlv = wl[...]
    b1v = b1[...]
    b2v = b2[...]

    def emb(lo1, hi1, lo2, hi2):
        return (jnp.dot(lo1 + b1v[:, :D], wlv[0:D], preferred_element_type=f32)
                + jnp.dot(hi1 + b1v[:, D:], wlv[D:2 * D], preferred_element_type=f32)
                + jnp.dot(lo2 + b2v[:, :D], wlv[2 * D:3 * D], preferred_element_type=f32)
                + jnp.dot(hi2 + b2v[:, D:], wlv[3 * D:], preferred_element_type=f32)
                + bl[...])

    u = emb(u1l[...].astype(f32), u1h[...].astype(f32),
            u2l[...].astype(f32), u2h[...].astype(f32))
    pv = emb(p1l[...].astype(f32), p1h[...].astype(f32),
             p2l[...].astype(f32), p2h[...].astype(f32))
    nv = emb(n1l[...].astype(f32).reshape(NEG * LB, D),
             n1h[...].astype(f32).reshape(NEG * LB, D),
             n2l[...].astype(f32).reshape(NEG * LB, D),
             n2h[...].astype(f32).reshape(NEG * LB, D))
    nv = nv.reshape(NEG, LB, 2 * D)
    pos_score = jnp.sum(u * pv, axis=1)
    neg_score = jnp.sum(nv * u[None, :, :], axis=2)
    diff = neg_score - pos_score[None, :] + 1.0
    part = jnp.sum(jnp.clip(jnp.mean(diff, axis=0), 1e-06, 10000.0))
    out_ref[...] = jnp.where(i == 0, 0.0, out_ref[...]) + part


def _loss_call(gs, wl, blr, b1r, b2r):
    urow = pl.BlockSpec((LB, D), lambda i: (i, 0))
    prow = pl.BlockSpec((LB, D), lambda i: (i, 0))
    nrow = pl.BlockSpec((NEG, LB, D), lambda i: (0, i, 0))
    return pl.pallas_call(
        _loss_body,
        grid=(NLB,),
        in_specs=([urow] * 4 + [prow] * 4 + [nrow] * 4
                  + [_rep((2 * KD, 2 * D)), _rep((1, 2 * D)),
                     _rep((1, KD)), _rep((1, KD))]),
        out_specs=pl.BlockSpec((1, 1), lambda i: (0, 0)),
        out_shape=jax.ShapeDtypeStruct((1, 1), f32),
    )(*gs, wl, blr, b1r, b2r)


def kernel(user, pos_item, neg_item, label, lookup_index, edge_index,
           edge_weight, word_emb, other_pos, user_pos, item_pos,
           W_ih0, W_hh0, b_ih0, b_hh0, W_ih1, W_hh1, b_ih1, b_hh1,
           W1, b1, W2, b2, Wl, bl):
    wt = word_emb.at[0].set(0.0)
    idx_pad = jnp.concatenate([
        lookup_index.reshape(-1).astype(i32),
        jnp.zeros((EMB_ROWS - N * T,), i32)])
    emb = _emb_gather(wt, idx_pad)
    emb3 = emb.reshape(NPAD, T, D)

    h1 = _gru_call(emb3, W_ih0.T, W_hh0.T, b_ih0.reshape(1, -1),
                   b_hh0.reshape(1, -1), W_ih1.T, W_hh1.T,
                   b_ih1.reshape(1, -1), b_hh1.reshape(1, -1))

    pos_pad = jnp.concatenate(
        [other_pos, user_pos, item_pos, jnp.zeros((NPAD - N, D), f32)], axis=0)
    sup1_lo, sup1_hi = _sup1_call(pos_pad, h1, W1)

    zpad = jnp.zeros((EPAD - E,), i32)
    srcp = jnp.concatenate([edge_index[0].astype(i32), zpad])
    dstp = jnp.concatenate([edge_index[1].astype(i32), zpad])
    wp = jnp.concatenate([edge_weight, jnp.zeros((EPAD - E,), f32)])

    perm = jnp.array([32 * (j // 32) + (j % 32) // 2 + 16 * (j % 2)
                      for j in range(D)], dtype=i32)
    b1p = jnp.concatenate([b1[:D][perm], b1[D:][perm]])
    b2p = jnp.concatenate([b2[:D][perm], b2[D:][perm]])
    W2p = jnp.concatenate([W2[:D][perm], W2[D:][perm]], axis=0)
    Wlp = jnp.concatenate([Wl[:D][perm], Wl[D:2 * D][perm],
                           Wl[2 * D:3 * D][perm], Wl[3 * D:][perm]], axis=0)

    a1lo, a1hi = _gcn_edges(sup1_lo, sup1_hi, srcp, dstp, wp)
    s2lo, s2hi = _sup2_call(a1lo, a1hi, W2p, b1p.reshape(1, -1))
    a2lo, a2hi = _gcn_edges(s2lo, s2hi, srcp, dstp, wp)

    gidx = jnp.concatenate([
        user + N_OTHER,
        pos_item + N_OTHER + N_USER,
        neg_item + N_OTHER + N_USER]).astype(i32)
    g1l, g1h, g2l, g2h = _final_gather(a1lo, a1hi, a2lo, a2hi, gidx)

    def parts(g):
        return g[:B], g[B:2 * B], g[2 * B:].reshape(NEG, B, D)

    u1l, p1l, n1l = parts(g1l)
    u1h, p1h, n1h = parts(g1h)
    u2l, p2l, n2l = parts(g2l)
    u2h, p2h, n2h = parts(g2h)
    gs = (u1l, u1h, u2l, u2h, p1l, p1h, p2l, p2h, n1l, n1h, n2l, n2h)
    out = _loss_call(gs, Wlp, bl.reshape(1, -1), b1p.reshape(1, -1),
                     b2p.reshape(1, -1))
    return out[0, 0]


# speedup vs baseline: 1.0571x; 1.0184x over previous
import functools

import jax
import jax.numpy as jnp
from jax import lax
from jax.experimental import pallas as pl
from jax.experimental.pallas import tpu as pltpu
from jax.experimental.pallas import tpu_sc as plsc

N_OTHER = 2000
N_USER = 20000
N_ITEM = 28000
N = N_OTHER + N_USER + N_ITEM
NPAD = 50048
E = 800000
EPAD = 819200
VOCAB = 100000
D = 64
KD = 128
T = 20
B = 4096
NEG = 5

NC = 2
NS = 16
NWK = NC * NS

f32 = jnp.float32
i32 = jnp.int32

_SC_MESH = dict(core_axis_name="c", subcore_axis_name="s")


EMB_ROWS = NPAD * T
EMB_PW = EMB_ROWS // NWK
EMB_G = 1360
EMB_CH = EMB_PW // EMB_G


@functools.partial(
    pl.kernel,
    mesh=plsc.VectorSubcoreMesh(**_SC_MESH),
    compiler_params=pltpu.CompilerParams(use_tc_tiling_on_sc=False),
    out_type=jax.ShapeDtypeStruct((EMB_ROWS, D), f32),
    scratch_types=[
        pltpu.VMEM((EMB_G,), i32),
        pltpu.VMEM((EMB_G, D), f32),
        pltpu.SemaphoreType.DMA,
    ],
)
def _emb_gather(tab, idx, out, idx_v, rows_v, sem):
    wid = lax.axis_index("s") * NC + lax.axis_index("c")

    def body(j, carry):
        base = wid * EMB_PW + j * EMB_G
        pltpu.sync_copy(idx.at[pl.ds(base, EMB_G)], idx_v)
        pltpu.async_copy(tab.at[idx_v], rows_v, sem).wait()
        pltpu.sync_copy(rows_v, out.at[pl.ds(base, EMB_G)])
        return carry

    lax.fori_loop(0, EMB_CH, body, 0)


NPASS = 2
RNG = NPAD // 2
NPO = NPASS * RNG
TRASH = RNG
ACCR = RNG + 8
GE = 320
CB = GE + 64
CBLK = CB // 64
PTE = EPAD // NS
GCH2 = PTE // (2 * GE)
CPT = RNG // NS
ZFULL = CPT // CB
ZREM = CPT - ZFULL * CB


def _gcn_scratch():
    return [
        pltpu.VMEM((3, GE), i32),
        pltpu.VMEM((CB,), i32),
        pltpu.VMEM((CB,), i32),
        pltpu.VMEM((CB,), f32),
        pltpu.VMEM((CBLK, D), i32),
        pltpu.VMEM((CB, D), f32),
        pltpu.VMEM((CB, D), jnp.bfloat16),
    ]


@functools.partial(
    pl.kernel,
    mesh=plsc.VectorSubcoreMesh(**_SC_MESH),
    compiler_params=pltpu.CompilerParams(
        use_tc_tiling_on_sc=False, needs_layout_passes=False),
    out_type=[
        jax.ShapeDtypeStruct((NPO, D), jnp.bfloat16),
        jax.ShapeDtypeStruct((NPO, D), jnp.bfloat16),
    ],
    scratch_types=_gcn_scratch() + _gcn_scratch() + [
        pltpu.VMEM_SHARED((ACCR, D), jnp.bfloat16),
        pltpu.SemaphoreType.DMA,
        pltpu.SemaphoreType.DMA,
        pltpu.SemaphoreType.DMA,
        pltpu.SemaphoreType.DMA,
        pltpu.SemaphoreType.DMA,
        pltpu.SemaphoreType.DMA,
    ],
)
def _gcn_edges(sup_lo, sup_hi, e3, out_lo, out_hi,
               e3A, csrcA, csidxA, cwA, crowA, rowsA, bfA,
               e3B, csrcB, csidxB, cwB, crowB, rowsB, bfB,
               acc, sgA, ssA, sgB, ssB, stA, stB):
    c = lax.axis_index("c")
    s = lax.axis_index("s")

    def half(sup, out):
        def stage(cid, e3_v, sem):
            return pltpu.make_async_copy(e3.at[cid], e3_v, sem)

        def compact(lo, e3_v, csrc_v, csidx_v, cw_v):
            def cg(g, cnt):
                s16 = e3_v[0, pl.ds(g * 16, 16)]
                d16 = e3_v[1, pl.ds(g * 16, 16)]
                w16 = plsc.bitcast(e3_v[2, pl.ds(g * 16, 16)], f32)
                l16 = d16 - lo
                m = (l16 >= 0) & (l16 < RNG)
                lane = lax.iota(i32, 16)
                key = jnp.where(m, lane, lane + 16)
                _, perm = plsc.sort_key_val(key, lane)
                csrc_v[pl.ds(cnt, 16)] = s16.at[perm].get(
                    mode="promise_in_bounds")
                csidx_v[pl.ds(cnt, 16)] = l16.at[perm].get(
                    mode="promise_in_bounds")
                cw_v[pl.ds(cnt, 16)] = w16.at[perm].get(
                    mode="promise_in_bounds")
                return cnt + jnp.sum(m.astype(i32))

            cnt = lax.fori_loop(0, GE // 16, cg, 0)
            for k in range(4):
                csrc_v[pl.ds(cnt + k * 16, 16)] = jnp.zeros((16,), i32)
                csidx_v[pl.ds(cnt + k * 16, 16)] = jnp.full((16,), TRASH, i32)
                cw_v[pl.ds(cnt + k * 16, 16)] = jnp.zeros((16,), f32)
            return (cnt + 63) // 64

        def gathers(nblk, csrc_v, rows_v, sem, start):
            for b in range(CBLK):
                cp = pltpu.make_async_copy(
                    sup.at[csrc_v.at[pl.ds(b * 64, 64)]],
                    rows_v.at[pl.ds(b * 64, 64)], sem)

                @pl.when(b < nblk)
                def _(cp=cp):
                    if start:
                        cp.start()
                    else:
                        cp.wait()

        def mul(nblk, cw_v, rows_v, bf_v):
            def mg(g, cr):
                w16 = cw_v[pl.ds(g * 16, 16)]
                for jj in range(16):
                    r = g * 16 + jj
                    wspl = w16.at[jnp.full((16,), jj, i32)].get(
                        mode="promise_in_bounds")
                    m0 = rows_v[r, pl.ds(0, 16)] * wspl
                    m1 = rows_v[r, pl.ds(16, 16)] * wspl
                    m2 = rows_v[r, pl.ds(32, 16)] * wspl
                    m3 = rows_v[r, pl.ds(48, 16)] * wspl
                    bf_v[r, pl.ds(0, 32)] = plsc.pack(
                        m0, m1, format=plsc.PackFormat.INTERLEAVED)
                    bf_v[r, pl.ds(32, 32)] = plsc.pack(
                        m2, m3, format=plsc.PackFormat.INTERLEAVED)
                return cr

            lax.fori_loop(0, nblk * 4, mg, 0)

        def scatters(nblk, csidx_v, crow_v, bf_v, sem, start):
            for b in range(CBLK):
                if start:
                    for k in range(4):
                        crow_v[b, pl.ds(k * 16, 16)] = (
                            csidx_v[pl.ds(b * 64 + k * 16, 16)])
                cp = pltpu.make_async_copy(
                    bf_v.at[pl.ds(b * 64, 64)],
                    acc.at[crow_v.at[b]], sem)

                @pl.when(b < nblk)
                def _(cp=cp):
                    if start:
                        cp.start(add=True)
                    else:
                        cp.wait()

        def do_pass(p, carry):
            lo = p * RNG

            def zb(r, cr):
                for k in range(2):
                    bfA[r, pl.ds(k * 32, 32)] = jnp.zeros((32,), jnp.bfloat16)
                return cr

            lax.fori_loop(0, CB, zb, 0)
            for q in range(ZFULL):
                pltpu.sync_copy(bfA.at[pl.ds(0, CB)],
                                acc.at[pl.ds(s * CPT + q * CB, CB)])
            pltpu.sync_copy(bfA.at[pl.ds(0, ZREM)],
                            acc.at[pl.ds(s * CPT + ZFULL * CB, ZREM)])
            plsc.subcore_barrier()

            def pair(i, cr):
                cidA = s * (PTE // GE) + 2 * i
                cpA = stage(cidA, e3A, stA)
                cpA.start()
                cpB = stage(cidA + 1, e3B, stB)
                cpB.start()
                cpA.wait()
                nA = compact(lo, e3A, csrcA, csidxA, cwA)
                gathers(nA, csrcA, rowsA, sgA, True)
                cpB.wait()
                nB = compact(lo, e3B, csrcB, csidxB, cwB)
                gathers(nB, csrcB, rowsB, sgB, True)
                gathers(nA, csrcA, rowsA, sgA, False)
                mul(nA, cwA, rowsA, bfA)
                scatters(nA, csidxA, crowA, bfA, ssA, True)
                gathers(nB, csrcB, rowsB, sgB, False)
                mul(nB, cwB, rowsB, bfB)
                scatters(nB, csidxB, crowB, bfB, ssB, True)
                scatters(nA, csidxA, crowA, bfA, ssA, False)
                scatters(nB, csidxB, crowB, bfB, ssB, False)
                return cr

            lax.fori_loop(0, GCH2, pair, 0)
            plsc.subcore_barrier()
            pltpu.sync_copy(acc.at[pl.ds(s * CPT, CPT)],
                            out.at[pl.ds(lo + s * CPT, CPT)])
            plsc.subcore_barrier()
            return carry

        lax.fori_loop(0, NPASS, do_pass, 0)

    @pl.when(c == 0)
    def _():
        half(sup_lo, out_lo)

    @pl.when(c == 1)
    def _():
        half(sup_hi, out_hi)


GB = (NEG + 2) * B
GPW = GB // NWK


@functools.partial(
    pl.kernel,
    mesh=plsc.VectorSubcoreMesh(**_SC_MESH),
    compiler_params=pltpu.CompilerParams(use_tc_tiling_on_sc=False),
    out_type=[jax.ShapeDtypeStruct((GB, D), jnp.bfloat16) for _ in range(4)],
    scratch_types=[
        pltpu.VMEM((GPW,), i32),
        pltpu.VMEM((GPW, D), jnp.bfloat16),
        pltpu.SemaphoreType.DMA,
    ],
)
def _final_gather(t1l, t1h, t2l, t2h, gidx, o1l, o1h, o2l, o2h,
                  idx_v, rows_v, sem):
    wid = lax.axis_index("s") * NC + lax.axis_index("c")
    base = wid * GPW
    pltpu.sync_copy(gidx.at[pl.ds(base, GPW)], idx_v)
    for tab, out in ((t1l, o1l), (t1h, o1h), (t2l, o2l), (t2h, o2h)):
        pltpu.async_copy(tab.at[idx_v], rows_v, sem).wait()
        pltpu.sync_copy(rows_v, out.at[pl.ds(base, GPW)])


BN = 1472
NB = NPAD // BN


def _gru_body(x_ref, wi0, wh0, bi0, bh0, wi1, wh1, bi1, bh1, out_ref, y0):
    x = x_ref[...]
    h = jnp.zeros((BN, D), f32)
    for t in range(T):
        gi = jnp.dot(x[:, t, :], wi0[...], preferred_element_type=f32) + bi0[...]
        gh = jnp.dot(h, wh0[...], preferred_element_type=f32) + bh0[...]
        r = jax.nn.sigmoid(gi[:, :D] + gh[:, :D])
        z = jax.nn.sigmoid(gi[:, D:2 * D] + gh[:, D:2 * D])
        n = jnp.tanh(gi[:, 2 * D:] + r * gh[:, 2 * D:])
        h = (1.0 - z) * n + z * h
        y0[t] = h
    h = jnp.zeros((BN, D), f32)
    for t in range(T):
        gi = jnp.dot(y0[t], wi1[...], preferred_element_type=f32) + bi1[...]
        gh = jnp.dot(h, wh1[...], preferred_element_type=f32) + bh1[...]
        r = jax.nn.sigmoid(gi[:, :D] + gh[:, :D])
        z = jax.nn.sigmoid(gi[:, D:2 * D] + gh[:, D:2 * D])
        n = jnp.tanh(gi[:, 2 * D:] + r * gh[:, 2 * D:])
        h = (1.0 - z) * n + z * h
    out_ref[...] = h


def _rep(shape):
    return pl.BlockSpec(shape, lambda i: tuple(0 for _ in shape))


def _gru_call(emb3, wi0, wh0, bi0, bh0, wi1, wh1, bi1, bh1):
    return pl.pallas_call(
        _gru_body,
        grid=(NB,),
        in_specs=[
            pl.BlockSpec((BN, T, D), lambda i: (i, 0, 0)),
            _rep((D, 3 * D)), _rep((D, 3 * D)), _rep((1, 3 * D)), _rep((1, 3 * D)),
            _rep((D, 3 * D)), _rep((D, 3 * D)), _rep((1, 3 * D)), _rep((1, 3 * D)),
        ],
        out_specs=pl.BlockSpec((BN, D), lambda i: (i, 0)),
        out_shape=jax.ShapeDtypeStruct((NPAD, D), f32),
        scratch_shapes=[pltpu.VMEM((T, BN, D), f32)],
    )(emb3, wi0, wh0, bi0, bh0, wi1, wh1, bi1, bh1)


def _sup1_body(pe_ref, h_ref, w_ref, olo_ref, ohi_ref):
    x = pe_ref[...] + h_ref[...]
    sup = jnp.dot(x, w_ref[...], preferred_element_type=f32)
    olo_ref[...] = sup[:, :D]
    ohi_ref[...] = sup[:, D:]


def _sup1_call(pos_pad, h1, w1):
    return pl.pallas_call(
        _sup1_body,
        grid=(NB,),
        in_specs=[
            pl.BlockSpec((BN, D), lambda i: (i, 0)),
            pl.BlockSpec((BN, D), lambda i: (i, 0)),
            _rep((D, KD)),
        ],
        out_specs=[pl.BlockSpec((BN, D), lambda i: (i, 0))] * 2,
        out_shape=[jax.ShapeDtypeStruct((NPAD, D), f32)] * 2,
    )(pos_pad, h1, w1)


def _sup2_body(alo_ref, ahi_ref, w_ref, b1_ref, olo_ref, ohi_ref):
    clo = alo_ref[...].astype(f32) + b1_ref[...][:, :D]
    chi = ahi_ref[...].astype(f32) + b1_ref[...][:, D:]
    sup = (jnp.dot(clo, w_ref[...][:D], preferred_element_type=f32)
           + jnp.dot(chi, w_ref[...][D:], preferred_element_type=f32))
    olo_ref[...] = sup[:, :D]
    ohi_ref[...] = sup[:, D:]


def _sup2_call(a1lo, a1hi, w2, b1r):
    return pl.pallas_call(
        _sup2_body,
        grid=(NB,),
        in_specs=[
            pl.BlockSpec((BN, D), lambda i: (i, 0)),
            pl.BlockSpec((BN, D), lambda i: (i, 0)),
            _rep((KD, KD)),
            _rep((1, KD)),
        ],
        out_specs=[pl.BlockSpec((BN, D), lambda i: (i, 0))] * 2,
        out_shape=[jax.ShapeDtypeStruct((NPAD, D), f32)] * 2,
    )(a1lo, a1hi, w2, b1r)


LB = 512
NLB = B // LB


def _loss_body(u1l, u1h, u2l, u2h, p1l, p1h, p2l, p2h,
               n1l, n1h, n2l, n2h, wl, bl, b1, b2, out_ref):
    i = pl.program_id(0)
    wlv = wl[...]
    b1v = b1[...]
    b2v = b2[...]

    def emb(lo1, hi1, lo2, hi2):
        return (jnp.dot(lo1 + b1v[:, :D], wlv[0:D], preferred_element_type=f32)
                + jnp.dot(hi1 + b1v[:, D:], wlv[D:2 * D], preferred_element_type=f32)
                + jnp.dot(lo2 + b2v[:, :D], wlv[2 * D:3 * D], preferred_element_type=f32)
                + jnp.dot(hi2 + b2v[:, D:], wlv[3 * D:], preferred_element_type=f32)
                + bl[...])

    u = emb(u1l[...].astype(f32), u1h[...].astype(f32),
            u2l[...].astype(f32), u2h[...].astype(f32))
    pv = emb(p1l[...].astype(f32), p1h[...].astype(f32),
             p2l[...].astype(f32), p2h[...].astype(f32))
    nv = emb(n1l[...].astype(f32).reshape(NEG * LB, D),
             n1h[...].astype(f32).reshape(NEG * LB, D),
             n2l[...].astype(f32).reshape(NEG * LB, D),
             n2h[...].astype(f32).reshape(NEG * LB, D))
    nv = nv.reshape(NEG, LB, 2 * D)
    pos_score = jnp.sum(u * pv, axis=1)
    neg_score = jnp.sum(nv * u[None, :, :], axis=2)
    diff = neg_score - pos_score[None, :] + 1.0
    part = jnp.sum(jnp.clip(jnp.mean(diff, axis=0), 1e-06, 10000.0))
    out_ref[...] = jnp.where(i == 0, 0.0, out_ref[...]) + part


def _loss_call(gs, wl, blr, b1r, b2r):
    urow = pl.BlockSpec((LB, D), lambda i: (i, 0))
    prow = pl.BlockSpec((LB, D), lambda i: (i, 0))
    nrow = pl.BlockSpec((NEG, LB, D), lambda i: (0, i, 0))
    return pl.pallas_call(
        _loss_body,
        grid=(NLB,),
        in_specs=([urow] * 4 + [prow] * 4 + [nrow] * 4
                  + [_rep((2 * KD, 2 * D)), _rep((1, 2 * D)),
                     _rep((1, KD)), _rep((1, KD))]),
        out_specs=pl.BlockSpec((1, 1), lambda i: (0, 0)),
        out_shape=jax.ShapeDtypeStruct((1, 1), f32),
    )(*gs, wl, blr, b1r, b2r)


def kernel(user, pos_item, neg_item, label, lookup_index, edge_index,
           edge_weight, word_emb, other_pos, user_pos, item_pos,
           W_ih0, W_hh0, b_ih0, b_hh0, W_ih1, W_hh1, b_ih1, b_hh1,
           W1, b1, W2, b2, Wl, bl):
    wt = word_emb.at[0].set(0.0)
    idx_pad = jnp.concatenate([
        lookup_index.reshape(-1).astype(i32),
        jnp.zeros((EMB_ROWS - N * T,), i32)])
    emb = _emb_gather(wt, idx_pad)
    emb3 = emb.reshape(NPAD, T, D)

    h1 = _gru_call(emb3, W_ih0.T, W_hh0.T, b_ih0.reshape(1, -1),
                   b_hh0.reshape(1, -1), W_ih1.T, W_hh1.T,
                   b_ih1.reshape(1, -1), b_hh1.reshape(1, -1))

    pos_pad = jnp.concatenate(
        [other_pos, user_pos, item_pos, jnp.zeros((NPAD - N, D), f32)], axis=0)
    sup1_lo, sup1_hi = _sup1_call(pos_pad, h1, W1)

    zpad = jnp.zeros((EPAD - E,), i32)
    srcp = jnp.concatenate([edge_index[0].astype(i32), zpad])
    dstp = jnp.concatenate([edge_index[1].astype(i32), zpad])
    wp = jnp.concatenate([edge_weight, jnp.zeros((EPAD - E,), f32)])
    nch = EPAD // GE
    e3 = jnp.stack([srcp.reshape(nch, GE), dstp.reshape(nch, GE),
                    jax.lax.bitcast_convert_type(wp, i32).reshape(nch, GE)],
                   axis=1)

    perm = jnp.array([32 * (j // 32) + (j % 32) // 2 + 16 * (j % 2)
                      for j in range(D)], dtype=i32)
    b1p = jnp.concatenate([b1[:D][perm], b1[D:][perm]])
    b2p = jnp.concatenate([b2[:D][perm], b2[D:][perm]])
    W2p = jnp.concatenate([W2[:D][perm], W2[D:][perm]], axis=0)
    Wlp = jnp.concatenate([Wl[:D][perm], Wl[D:2 * D][perm],
                           Wl[2 * D:3 * D][perm], Wl[3 * D:][perm]], axis=0)

    a1lo, a1hi = _gcn_edges(sup1_lo, sup1_hi, e3)
    s2lo, s2hi = _sup2_call(a1lo, a1hi, W2p, b1p.reshape(1, -1))
    a2lo, a2hi = _gcn_edges(s2lo, s2hi, e3)

    gidx = jnp.concatenate([
        user + N_OTHER,
        pos_item + N_OTHER + N_USER,
        neg_item + N_OTHER + N_USER]).astype(i32)
    g1l, g1h, g2l, g2h = _final_gather(a1lo, a1hi, a2lo, a2hi, gidx)

    def parts(g):
        return g[:B], g[B:2 * B], g[2 * B:].reshape(NEG, B, D)

    u1l, p1l, n1l = parts(g1l)
    u1h, p1h, n1h = parts(g1h)
    u2l, p2l, n2l = parts(g2l)
    u2h, p2h, n2h = parts(g2h)
    gs = (u1l, u1h, u2l, u2h, p1l, p1h, p2l, p2h, n1l, n1h, n2l, n2h)
    out = _loss_call(gs, Wlp, bl.reshape(1, -1), b1p.reshape(1, -1),
                     b2p.reshape(1, -1))
    return out[0, 0]


# speedup vs baseline: 1.0577x; 1.0006x over previous
import functools

import jax
import jax.numpy as jnp
from jax import lax
from jax.experimental import pallas as pl
from jax.experimental.pallas import tpu as pltpu
from jax.experimental.pallas import tpu_sc as plsc

N_OTHER = 2000
N_USER = 20000
N_ITEM = 28000
N = N_OTHER + N_USER + N_ITEM
NPAD = 50048
E = 800000
EPAD = 819200
VOCAB = 100000
D = 64
KD = 128
T = 20
B = 4096
NEG = 5

NC = 2
NS = 16
NWK = NC * NS

f32 = jnp.float32
i32 = jnp.int32

_SC_MESH = dict(core_axis_name="c", subcore_axis_name="s")


EMB_ROWS = NPAD * T
EMB_PW = EMB_ROWS // NWK
EMB_G = 1360
EMB_CH = EMB_PW // EMB_G


@functools.partial(
    pl.kernel,
    mesh=plsc.VectorSubcoreMesh(**_SC_MESH),
    compiler_params=pltpu.CompilerParams(use_tc_tiling_on_sc=False),
    out_type=jax.ShapeDtypeStruct((EMB_ROWS, D), f32),
    scratch_types=[
        pltpu.VMEM((EMB_G,), i32),
        pltpu.VMEM((EMB_G, D), f32),
        pltpu.SemaphoreType.DMA,
    ],
)
def _emb_gather(tab, idx, out, idx_v, rows_v, sem):
    wid = lax.axis_index("s") * NC + lax.axis_index("c")

    def body(j, carry):
        base = wid * EMB_PW + j * EMB_G
        pltpu.sync_copy(idx.at[pl.ds(base, EMB_G)], idx_v)
        pltpu.async_copy(tab.at[idx_v], rows_v, sem).wait()
        pltpu.sync_copy(rows_v, out.at[pl.ds(base, EMB_G)])
        return carry

    lax.fori_loop(0, EMB_CH, body, 0)


NPASS = 2
RNG = NPAD // 2
NPO = NPASS * RNG
TRASH = RNG
ACCR = RNG + 8
GE = 320
CB = GE + 64
CBLK = CB // 64
PTE = EPAD // NS
GCH2 = PTE // (2 * GE)
CPT = RNG // NS
ZFULL = CPT // CB
ZREM = CPT - ZFULL * CB


def _gcn_scratch():
    return [
        pltpu.VMEM((3, GE), i32),
        pltpu.VMEM((CB,), i32),
        pltpu.VMEM((CB,), i32),
        pltpu.VMEM((CB,), f32),
        pltpu.VMEM((CBLK, D), i32),
        pltpu.VMEM((CB, D), f32),
        pltpu.VMEM((CB, D), jnp.bfloat16),
    ]


@functools.partial(
    pl.kernel,
    mesh=plsc.VectorSubcoreMesh(**_SC_MESH),
    compiler_params=pltpu.CompilerParams(
        use_tc_tiling_on_sc=False, needs_layout_passes=False),
    out_type=[
        jax.ShapeDtypeStruct((NPO, D), jnp.bfloat16),
        jax.ShapeDtypeStruct((NPO, D), jnp.bfloat16),
    ],
    scratch_types=_gcn_scratch() + _gcn_scratch() + [
        pltpu.VMEM_SHARED((ACCR, D), jnp.bfloat16),
        pltpu.SemaphoreType.DMA,
        pltpu.SemaphoreType.DMA,
        pltpu.SemaphoreType.DMA,
        pltpu.SemaphoreType.DMA,
        pltpu.SemaphoreType.DMA,
        pltpu.SemaphoreType.DMA,
    ],
)
def _gcn_edges(sup_lo, sup_hi, e3, out_lo, out_hi,
               e3A, csrcA, csidxA, cwA, crowA, rowsA, bfA,
               e3B, csrcB, csidxB, cwB, crowB, rowsB, bfB,
               acc, sgA, ssA, sgB, ssB, stA, stB):
    c = lax.axis_index("c")
    s = lax.axis_index("s")

    def half(sup, out):
        def stage(cid, e3_v, sem):
            return pltpu.make_async_copy(e3.at[cid], e3_v, sem)

        def compact(lo, e3_v, csrc_v, csidx_v, cw_v):
            def cg(g, cnt):
                s16 = e3_v[0, pl.ds(g * 16, 16)]
                d16 = e3_v[1, pl.ds(g * 16, 16)]
                w16 = plsc.bitcast(e3_v[2, pl.ds(g * 16, 16)], f32)
                l16 = d16 - lo
                m = (l16 >= 0) & (l16 < RNG)
                lane = lax.iota(i32, 16)
                key = jnp.where(m, lane, lane + 16)
                _, perm = plsc.sort_key_val(key, lane)
                csrc_v[pl.ds(cnt, 16)] = s16.at[perm].get(
                    mode="promise_in_bounds")
                csidx_v[pl.ds(cnt, 16)] = l16.at[perm].get(
                    mode="promise_in_bounds")
                cw_v[pl.ds(cnt, 16)] = w16.at[perm].get(
                    mode="promise_in_bounds")
                return cnt + jnp.sum(m.astype(i32))

            cnt = lax.fori_loop(0, GE // 16, cg, 0)
            for k in range(4):
                csrc_v[pl.ds(cnt + k * 16, 16)] = jnp.zeros((16,), i32)
                csidx_v[pl.ds(cnt + k * 16, 16)] = jnp.full((16,), TRASH, i32)
                cw_v[pl.ds(cnt + k * 16, 16)] = jnp.zeros((16,), f32)
            return (cnt + 63) // 64

        def gathers(nblk, csrc_v, rows_v, sem, start):
            for b in range(CBLK):
                cp = pltpu.make_async_copy(
                    sup.at[csrc_v.at[pl.ds(b * 64, 64)]],
                    rows_v.at[pl.ds(b * 64, 64)], sem)

                @pl.when(b < nblk)
                def _(cp=cp):
                    if start:
                        cp.start()
                    else:
                        cp.wait()

        def mul(nblk, cw_v, rows_v, bf_v):
            def mg(g, cr):
                w16 = cw_v[pl.ds(g * 16, 16)]
                for jj in range(16):
                    r = g * 16 + jj
                    wspl = w16.at[jnp.full((16,), jj, i32)].get(
                        mode="promise_in_bounds")
                    m0 = rows_v[r, pl.ds(0, 16)] * wspl
                    m1 = rows_v[r, pl.ds(16, 16)] * wspl
                    m2 = rows_v[r, pl.ds(32, 16)] * wspl
                    m3 = rows_v[r, pl.ds(48, 16)] * wspl
                    bf_v[r, pl.ds(0, 32)] = plsc.pack(
                        m0, m1, format=plsc.PackFormat.INTERLEAVED)
                    bf_v[r, pl.ds(32, 32)] = plsc.pack(
                        m2, m3, format=plsc.PackFormat.INTERLEAVED)
                return cr

            lax.fori_loop(0, nblk * 4, mg, 0)

        def scatters(nblk, csidx_v, crow_v, bf_v, sem, start):
            for b in range(CBLK):
                if start:
                    for k in range(4):
                        crow_v[b, pl.ds(k * 16, 16)] = (
                            csidx_v[pl.ds(b * 64 + k * 16, 16)])
                cp = pltpu.make_async_copy(
                    bf_v.at[pl.ds(b * 64, 64)],
                    acc.at[crow_v.at[b]], sem)

                @pl.when(b < nblk)
                def _(cp=cp):
                    if start:
                        cp.start(add=True)
                    else:
                        cp.wait()

        def do_pass(p, carry):
            lo = p * RNG

            def zb(r, cr):
                for k in range(2):
                    bfA[r, pl.ds(k * 32, 32)] = jnp.zeros((32,), jnp.bfloat16)
                return cr

            lax.fori_loop(0, CB, zb, 0)
            for q in range(ZFULL):
                pltpu.sync_copy(bfA.at[pl.ds(0, CB)],
                                acc.at[pl.ds(s * CPT + q * CB, CB)])
            pltpu.sync_copy(bfA.at[pl.ds(0, ZREM)],
                            acc.at[pl.ds(s * CPT + ZFULL * CB, ZREM)])
            plsc.subcore_barrier()

            cid0 = s * (PTE // GE)
            stage(cid0, e3A, stA).start()
            stage(cid0 + 1, e3B, stB).start()

            def pair(i, cr):
                cidA = cid0 + 2 * i
                stage(cidA, e3A, stA).wait()
                nA = compact(lo, e3A, csrcA, csidxA, cwA)
                gathers(nA, csrcA, rowsA, sgA, True)
                stage(cidA + 1, e3B, stB).wait()
                nB = compact(lo, e3B, csrcB, csidxB, cwB)
                gathers(nB, csrcB, rowsB, sgB, True)

                @pl.when(i + 1 < GCH2)
                def _():
                    stage(cidA + 2, e3A, stA).start()
                    stage(cidA + 3, e3B, stB).start()

                gathers(nA, csrcA, rowsA, sgA, False)
                mul(nA, cwA, rowsA, bfA)
                scatters(nA, csidxA, crowA, bfA, ssA, True)
                gathers(nB, csrcB, rowsB, sgB, False)
                mul(nB, cwB, rowsB, bfB)
                scatters(nB, csidxB, crowB, bfB, ssB, True)
                scatters(nA, csidxA, crowA, bfA, ssA, False)
                scatters(nB, csidxB, crowB, bfB, ssB, False)
                return cr

            lax.fori_loop(0, GCH2, pair, 0)
            plsc.subcore_barrier()
            pltpu.sync_copy(acc.at[pl.ds(s * CPT, CPT)],
                            out.at[pl.ds(lo + s * CPT, CPT)])
            plsc.subcore_barrier()
            return carry

        lax.fori_loop(0, NPASS, do_pass, 0)

    @pl.when(c == 0)
    def _():
        half(sup_lo, out_lo)

    @pl.when(c == 1)
    def _():
        half(sup_hi, out_hi)


GB = (NEG + 2) * B
GPW = GB // NWK


@functools.partial(
    pl.kernel,
    mesh=plsc.VectorSubcoreMesh(**_SC_MESH),
    compiler_params=pltpu.CompilerParams(use_tc_tiling_on_sc=False),
    out_type=[jax.ShapeDtypeStruct((GB, D), jnp.bfloat16) for _ in range(4)],
    scratch_types=[
        pltpu.VMEM((GPW,), i32),
        pltpu.VMEM((GPW, D), jnp.bfloat16),
        pltpu.SemaphoreType.DMA,
    ],
)
def _final_gather(t1l, t1h, t2l, t2h, gidx, o1l, o1h, o2l, o2h,
                  idx_v, rows_v, sem):
    wid = lax.axis_index("s") * NC + lax.axis_index("c")
    base = wid * GPW
    pltpu.sync_copy(gidx.at[pl.ds(base, GPW)], idx_v)
    for tab, out in ((t1l, o1l), (t1h, o1h), (t2l, o2l), (t2h, o2h)):
        pltpu.async_copy(tab.at[idx_v], rows_v, sem).wait()
        pltpu.sync_copy(rows_v, out.at[pl.ds(base, GPW)])


BN = 1472
NB = NPAD // BN


def _gru_body(x_ref, wi0, wh0, bi0, bh0, wi1, wh1, bi1, bh1, out_ref, y0):
    x = x_ref[...]
    h = jnp.zeros((BN, D), f32)
    for t in range(T):
        gi = jnp.dot(x[:, t, :], wi0[...], preferred_element_type=f32) + bi0[...]
        gh = jnp.dot(h, wh0[...], preferred_element_type=f32) + bh0[...]
        r = jax.nn.sigmoid(gi[:, :D] + gh[:, :D])
        z = jax.nn.sigmoid(gi[:, D:2 * D] + gh[:, D:2 * D])
        n = jnp.tanh(gi[:, 2 * D:] + r * gh[:, 2 * D:])
        h = (1.0 - z) * n + z * h
        y0[t] = h
    h = jnp.zeros((BN, D), f32)
    for t in range(T):
        gi = jnp.dot(y0[t], wi1[...], preferred_element_type=f32) + bi1[...]
        gh = jnp.dot(h, wh1[...], preferred_element_type=f32) + bh1[...]
        r = jax.nn.sigmoid(gi[:, :D] + gh[:, :D])
        z = jax.nn.sigmoid(gi[:, D:2 * D] + gh[:, D:2 * D])
        n = jnp.tanh(gi[:, 2 * D:] + r * gh[:, 2 * D:])
        h = (1.0 - z) * n + z * h
    out_ref[...] = h


def _rep(shape):
    return pl.BlockSpec(shape, lambda i: tuple(0 for _ in shape))


def _gru_call(emb3, wi0, wh0, bi0, bh0, wi1, wh1, bi1, bh1):
    return pl.pallas_call(
        _gru_body,
        grid=(NB,),
        in_specs=[
            pl.BlockSpec((BN, T, D), lambda i: (i, 0, 0)),
            _rep((D, 3 * D)), _rep((D, 3 * D)), _rep((1, 3 * D)), _rep((1, 3 * D)),
            _rep((D, 3 * D)), _rep((D, 3 * D)), _rep((1, 3 * D)), _rep((1, 3 * D)),
        ],
        out_specs=pl.BlockSpec((BN, D), lambda i: (i, 0)),
        out_shape=jax.ShapeDtypeStruct((NPAD, D), f32),
        scratch_shapes=[pltpu.VMEM((T, BN, D), f32)],
    )(emb3, wi0, wh0, bi0, bh0, wi1, wh1, bi1, bh1)


def _sup1_body(pe_ref, h_ref, w_ref, olo_ref, ohi_ref):
    x = pe_ref[...] + h_ref[...]
    sup = jnp.dot(x, w_ref[...], preferred_element_type=f32)
    olo_ref[...] = sup[:, :D]
    ohi_ref[...] = sup[:, D:]


def _sup1_call(pos_pad, h1, w1):
    return pl.pallas_call(
        _sup1_body,
        grid=(NB,),
        in_specs=[
            pl.BlockSpec((BN, D), lambda i: (i, 0)),
            pl.BlockSpec((BN, D), lambda i: (i, 0)),
            _rep((D, KD)),
        ],
        out_specs=[pl.BlockSpec((BN, D), lambda i: (i, 0))] * 2,
        out_shape=[jax.ShapeDtypeStruct((NPAD, D), f32)] * 2,
    )(pos_pad, h1, w1)


def _sup2_body(alo_ref, ahi_ref, w_ref, b1_ref, olo_ref, ohi_ref):
    clo = alo_ref[...].astype(f32) + b1_ref[...][:, :D]
    chi = ahi_ref[...].astype(f32) + b1_ref[...][:, D:]
    sup = (jnp.dot(clo, w_ref[...][:D], preferred_element_type=f32)
           + jnp.dot(chi, w_ref[...][D:], preferred_element_type=f32))
    olo_ref[...] = sup[:, :D]
    ohi_ref[...] = sup[:, D:]


def _sup2_call(a1lo, a1hi, w2, b1r):
    return pl.pallas_call(
        _sup2_body,
        grid=(NB,),
        in_specs=[
            pl.BlockSpec((BN, D), lambda i: (i, 0)),
            pl.BlockSpec((BN, D), lambda i: (i, 0)),
            _rep((KD, KD)),
            _rep((1, KD)),
        ],
        out_specs=[pl.BlockSpec((BN, D), lambda i: (i, 0))] * 2,
        out_shape=[jax.ShapeDtypeStruct((NPAD, D), f32)] * 2,
    )(a1lo, a1hi, w2, b1r)


LB = 512
NLB = B // LB


def _loss_body(u1l, u1h, u2l, u2h, p1l, p1h, p2l, p2h,
               n1l, n1h, n2l, n2h, wl, bl, b1, b2, out_ref):
    i = pl.program_id(0)
    wlv = wl[...]
    b1v = b1[...]
    b2v = b2[...]

    def emb(lo1, hi1, lo2, hi2):
        return (jnp.dot(lo1 + b1v[:, :D], wlv[0:D], preferred_element_type=f32)
                + jnp.dot(hi1 + b1v[:, D:], wlv[D:2 * D], preferred_element_type=f32)
                + jnp.dot(lo2 + b2v[:, :D], wlv[2 * D:3 * D], preferred_element_type=f32)
                + jnp.dot(hi2 + b2v[:, D:], wlv[3 * D:], preferred_element_type=f32)
                + bl[...])

    u = emb(u1l[...].astype(f32), u1h[...].astype(f32),
            u2l[...].astype(f32), u2h[...].astype(f32))
    pv = emb(p1l[...].astype(f32), p1h[...].astype(f32),
             p2l[...].astype(f32), p2h[...].astype(f32))
    nv = emb(n1l[...].astype(f32).reshape(NEG * LB, D),
             n1h[...].astype(f32).reshape(NEG * LB, D),
             n2l[...].astype(f32).reshape(NEG * LB, D),
             n2h[...].astype(f32).reshape(NEG * LB, D))
    nv = nv.reshape(NEG, LB, 2 * D)
    pos_score = jnp.sum(u * pv, axis=1)
    neg_score = jnp.sum(nv * u[None, :, :], axis=2)
    diff = neg_score - pos_score[None, :] + 1.0
    part = jnp.sum(jnp.clip(jnp.mean(diff, axis=0), 1e-06, 10000.0))
    out_ref[...] = jnp.where(i == 0, 0.0, out_ref[...]) + part


def _loss_call(gs, wl, blr, b1r, b2r):
    urow = pl.BlockSpec((LB, D), lambda i: (i, 0))
    prow = pl.BlockSpec((LB, D), lambda i: (i, 0))
    nrow = pl.BlockSpec((NEG, LB, D), lambda i: (0, i, 0))
    return pl.pallas_call(
        _loss_body,
        grid=(NLB,),
        in_specs=([urow] * 4 + [prow] * 4 + [nrow] * 4
                  + [_rep((2 * KD, 2 * D)), _rep((1, 2 * D)),
                     _rep((1, KD)), _rep((1, KD))]),
        out_specs=pl.BlockSpec((1, 1), lambda i: (0, 0)),
        out_shape=jax.ShapeDtypeStruct((1, 1), f32),
    )(*gs, wl, blr, b1r, b2r)


def kernel(user, pos_item, neg_item, label, lookup_index, edge_index,
           edge_weight, word_emb, other_pos, user_pos, item_pos,
           W_ih0, W_hh0, b_ih0, b_hh0, W_ih1, W_hh1, b_ih1, b_hh1,
           W1, b1, W2, b2, Wl, bl):
    wt = word_emb.at[0].set(0.0)
    idx_pad = jnp.concatenate([
        lookup_index.reshape(-1).astype(i32),
        jnp.zeros((EMB_ROWS - N * T,), i32)])
    emb = _emb_gather(wt, idx_pad)
    emb3 = emb.reshape(NPAD, T, D)

    h1 = _gru_call(emb3, W_ih0.T, W_hh0.T, b_ih0.reshape(1, -1),
                   b_hh0.reshape(1, -1), W_ih1.T, W_hh1.T,
                   b_ih1.reshape(1, -1), b_hh1.reshape(1, -1))

    pos_pad = jnp.concatenate(
        [other_pos, user_pos, item_pos, jnp.zeros((NPAD - N, D), f32)], axis=0)
    sup1_lo, sup1_hi = _sup1_call(pos_pad, h1, W1)

    zpad = jnp.zeros((EPAD - E,), i32)
    srcp = jnp.concatenate([edge_index[0].astype(i32), zpad])
    dstp = jnp.concatenate([edge_index[1].astype(i32), zpad])
    wp = jnp.concatenate([edge_weight, jnp.zeros((EPAD - E,), f32)])
    nch = EPAD // GE
    e3 = jnp.stack([srcp.reshape(nch, GE), dstp.reshape(nch, GE),
                    jax.lax.bitcast_convert_type(wp, i32).reshape(nch, GE)],
                   axis=1)

    perm = jnp.array([32 * (j // 32) + (j % 32) // 2 + 16 * (j % 2)
                      for j in range(D)], dtype=i32)
    b1p = jnp.concatenate([b1[:D][perm], b1[D:][perm]])
    b2p = jnp.concatenate([b2[:D][perm], b2[D:][perm]])
    W2p = jnp.concatenate([W2[:D][perm], W2[D:][perm]], axis=0)
    Wlp = jnp.concatenate([Wl[:D][perm], Wl[D:2 * D][perm],
                           Wl[2 * D:3 * D][perm], Wl[3 * D:][perm]], axis=0)

    a1lo, a1hi = _gcn_edges(sup1_lo, sup1_hi, e3)
    s2lo, s2hi = _sup2_call(a1lo, a1hi, W2p, b1p.reshape(1, -1))
    a2lo, a2hi = _gcn_edges(s2lo, s2hi, e3)

    gidx = jnp.concatenate([
        user + N_OTHER,
        pos_item + N_OTHER + N_USER,
        neg_item + N_OTHER + N_USER]).astype(i32)
    g1l, g1h, g2l, g2h = _final_gather(a1lo, a1hi, a2lo, a2hi, gidx)

    def parts(g):
        return g[:B], g[B:2 * B], g[2 * B:].reshape(NEG, B, D)

    u1l, p1l, n1l = parts(g1l)
    u1h, p1h, n1h = parts(g1h)
    u2l, p2l, n2l = parts(g2l)
    u2h, p2h, n2h = parts(g2h)
    gs = (u1l, u1h, u2l, u2h, p1l, p1h, p2l, p2h, n1l, n1h, n2l, n2h)
    out = _loss_call(gs, Wlp, bl.reshape(1, -1), b1p.reshape(1, -1),
                     b2p.reshape(1, -1))
    return out[0, 0]
